# baseline (speedup 1.0000x reference)
import functools

import jax
import jax.numpy as jnp
from jax import lax
from jax.experimental import pallas as pl
from jax.experimental.pallas import tpu as pltpu
from jax.experimental.pallas import tpu_sc as plsc

N = 10000
D = 128
G = 64
NC = 2
NS = 16
NW = NC * NS
E_CHUNK = 128
N_PAD = 10240
ROWS_PER_TILE = N_PAD // NS
ROW_BLK = 1000
N_BLKS = N // ROW_BLK


def _combine_body(p_ref, h_ref, wrel_ref, wroot_ref, b_ref, out_ref):
    agg = p_ref[0] + p_ref[1]
    out = (jnp.dot(agg, wrel_ref[...], preferred_element_type=jnp.float32)
           + jnp.dot(h_ref[...], wroot_ref[...], preferred_element_type=jnp.float32)
           + b_ref[...])
    out_ref[...] = jnp.maximum(out, 0.0)


def _combine_matmul(p, h, wrel, wroot, b_row):
    return pl.pallas_call(
        _combine_body,
        grid=(N_BLKS,),
        in_specs=[
            pl.BlockSpec((NC, ROW_BLK, D), lambda i: (0, i, 0)),
            pl.BlockSpec((ROW_BLK, D), lambda i: (i, 0)),
            pl.BlockSpec((D, D), lambda i: (0, 0)),
            pl.BlockSpec((D, D), lambda i: (0, 0)),
            pl.BlockSpec((1, D), lambda i: (0, 0)),
        ],
        out_specs=pl.BlockSpec((ROW_BLK, D), lambda i: (i, 0)),
        out_shape=jax.ShapeDtypeStruct((N, D), jnp.float32),
    )(p, h, wrel, wroot, b_row)


def _final_body(p_ref, h_ref, wrel_ref, wroot_ref, b3_ref, batch_ref,
                wlin_ref, blin_ref, out_ref, sums, counts):
    i = pl.program_id(0)

    @pl.when(i == 0)
    def _():
        sums[...] = jnp.zeros_like(sums)
        counts[...] = jnp.zeros_like(counts)

    agg = p_ref[0] + p_ref[1]
    h = (jnp.dot(agg, wrel_ref[...], preferred_element_type=jnp.float32)
         + jnp.dot(h_ref[...], wroot_ref[...], preferred_element_type=jnp.float32)
         + b3_ref[...])
    b = batch_ref[0]
    gid = lax.broadcasted_iota(jnp.int32, (G, ROW_BLK), 0)
    onehot = (gid == b).astype(jnp.float32)
    sums[...] += jnp.dot(onehot, h, preferred_element_type=jnp.float32,
                         precision=lax.Precision.HIGHEST)
    counts[...] = counts[...] + jnp.sum(onehot, axis=1, keepdims=True)

    @pl.when(i == pl.num_programs(0) - 1)
    def _():
        pooled = sums[...] / jnp.maximum(counts[...], 1.0)
        out_ref[...] = jnp.dot(pooled, wlin_ref[...],
                               preferred_element_type=jnp.float32,
                               precision=lax.Precision.HIGHEST) + blin_ref[...]


def _final_pool(p, h, wrel, wroot, b3_row, batch3, wlin_pad, blin_row):
    return pl.pallas_call(
        _final_body,
        grid=(N_BLKS,),
        in_specs=[
            pl.BlockSpec((NC, ROW_BLK, D), lambda i: (0, i, 0)),
            pl.BlockSpec((ROW_BLK, D), lambda i: (i, 0)),
            pl.BlockSpec((D, D), lambda i: (0, 0)),
            pl.BlockSpec((D, D), lambda i: (0, 0)),
            pl.BlockSpec((1, D), lambda i: (0, 0)),
            pl.BlockSpec((1, 1, ROW_BLK), lambda i: (i, 0, 0)),
            pl.BlockSpec((D, D), lambda i: (0, 0)),
            pl.BlockSpec((1, D), lambda i: (0, 0)),
        ],
        out_specs=pl.BlockSpec((G, D), lambda i: (0, 0)),
        out_shape=jax.ShapeDtypeStruct((G, D), jnp.float32),
        scratch_shapes=[
            pltpu.VMEM((G, D), jnp.float32),
            pltpu.VMEM((G, D), jnp.float32),
        ],
        compiler_params=pltpu.CompilerParams(
            dimension_semantics=("arbitrary",)),
    )(p, h, wrel, wroot, b3_row, batch3, wlin_pad, blin_row)


def _sc_scatter(y, src, dst, zeros):
    E = src.shape[0]
    epw = E // NW
    n_chunks = epw // E_CHUNK
    tail = epw % E_CHUNK

    assert n_chunks >= 4 and (n_chunks - 2) % 4 == 0

    mesh = plsc.VectorSubcoreMesh(
        core_axis_name="c", subcore_axis_name="s",
        num_cores=NC, num_subcores=NS)

    @functools.partial(
        pl.kernel,
        out_type=jax.ShapeDtypeStruct((NC * N_PAD, D), jnp.float32),
        mesh=mesh,
        scratch_types=(
            [pltpu.VMEM((E_CHUNK,), jnp.int32)] * 8
            + [pltpu.VMEM((E_CHUNK, D), jnp.float32)] * 2
            + [pltpu.VMEM((max(tail, 8),), jnp.int32)] * 2
            + [pltpu.VMEM_SHARED((N_PAD, D), jnp.float32)]
            + [pltpu.SemaphoreType.DMA] * 6
        ),
    )
    def k(y_hbm, src_hbm, dst_hbm, zeros_hbm, out_hbm,
          s0, s1, s2, s3, d0, d1, d2, d3, r0, r1,
          src_t, dst_t, acc, i0, i1, i2, i3, g0, g1):
        srcs = [s0, s1, s2, s3]
        dsts = [d0, d1, d2, d3]
        rows = [r0, r1]
        si = [i0, i1, i2, i3]
        sg = [g0, g1]
        c = lax.axis_index("c")
        s = lax.axis_index("s")
        wid = c * NS + s
        row0 = s * ROWS_PER_TILE

        pltpu.sync_copy(zeros_hbm, acc.at[pl.ds(row0, ROWS_PER_TILE)])
        plsc.subcore_barrier()

        base0 = wid * epw

        def idx_start(i, u):
            b = base0 + i * E_CHUNK
            pltpu.async_copy(src_hbm.at[pl.ds(b, E_CHUNK)], srcs[u], si[u])
            pltpu.async_copy(dst_hbm.at[pl.ds(b, E_CHUNK)], dsts[u], si[u])

        def idx_wait(u):
            pltpu.make_async_copy(src_hbm.at[pl.ds(0, E_CHUNK)], srcs[u], si[u]).wait()
            pltpu.make_async_copy(dst_hbm.at[pl.ds(0, E_CHUNK)], dsts[u], si[u]).wait()

        def g_start(u, b):
            pltpu.async_copy(y_hbm.at[srcs[u]], rows[b], sg[b])

        def g_wait(u, b):
            pltpu.make_async_copy(y_hbm.at[srcs[u]], rows[b], sg[b]).wait()

        def scat(u, b):
            pltpu.sync_copy(rows[b], acc.at[dsts[u]], add=True)

        def stage(i, u, b, first, guard_ok):
            idx_wait(u)
            g_start(u, b)
            if not first:
                g_wait((u - 1) % 4, 1 - b)
                scat((u - 1) % 4, 1 - b)
            if guard_ok is None:
                idx_start(i + 2, (u + 2) % 4)
            elif guard_ok is not False:
                @pl.when(guard_ok)
                def _():
                    idx_start(i + 2, (u + 2) % 4)

        idx_start(0, 0)
        idx_start(1, 1)
        stage(0, 0, 0, True, None)
        stage(1, 1, 1, False, None)

        def body(q, carry):
            i0_ = 2 + 4 * q
            for u in range(4):
                i = i0_ + u
                ub = (2 + u) % 4
                stage(i, ub, u % 2, False, i + 2 < n_chunks)
            return carry

        lax.fori_loop(0, (n_chunks - 2) // 4, body, 0)

        g_wait((n_chunks - 1) % 4, (n_chunks - 1) % 2)
        scat((n_chunks - 1) % 4, (n_chunks - 1) % 2)

        if tail:
            tb = base0 + n_chunks * E_CHUNK
            pltpu.sync_copy(src_hbm.at[pl.ds(tb, tail)], src_t)
            pltpu.sync_copy(dst_hbm.at[pl.ds(tb, tail)], dst_t)
            rows_t = rows[0].at[pl.ds(0, tail)]
            pltpu.async_copy(y_hbm.at[src_t], rows_t, sg[0]).wait()
            pltpu.sync_copy(rows_t, acc.at[dst_t], add=True)

        plsc.subcore_barrier()

        pltpu.sync_copy(acc.at[pl.ds(row0, ROWS_PER_TILE)],
                        out_hbm.at[pl.ds(c * N_PAD + row0, ROWS_PER_TILE)])

    return k(y, src, dst, zeros)


def kernel(x, edge_index, batch, W1_rel, b1, W1_root, W2_rel, b2, W2_root,
           W3_rel, b3, W3_root, Wlin, blin):
    src = edge_index[0]
    dst = edge_index[1]
    zeros = jnp.zeros((ROWS_PER_TILE, D), jnp.float32)
    b1r = b1.reshape(1, D)
    b2r = b2.reshape(1, D)
    b3r = b3.reshape(1, D)
    batch3 = batch.reshape(N_BLKS, 1, ROW_BLK)
    wlin_pad = jnp.zeros((D, D), jnp.float32).at[:, : Wlin.shape[1]].set(Wlin)
    blin_row = jnp.zeros((1, D), jnp.float32).at[0, : blin.shape[0]].set(blin)

    p1 = _sc_scatter(x, src, dst, zeros).reshape(NC, N_PAD, D)
    h1 = _combine_matmul(p1, x, W1_rel, W1_root, b1r)
    p2 = _sc_scatter(h1, src, dst, zeros).reshape(NC, N_PAD, D)
    h2 = _combine_matmul(p2, h1, W2_rel, W2_root, b2r)
    p3 = _sc_scatter(h2, src, dst, zeros).reshape(NC, N_PAD, D)
    out = _final_pool(p3, h2, W3_rel, W3_root, b3r, batch3, wlin_pad, blin_row)
    return out[:, : Wlin.shape[1]]


# speedup vs baseline: 1.0011x; 1.0011x over previous
import functools

import jax
import jax.numpy as jnp
from jax import lax
from jax.experimental import pallas as pl
from jax.experimental.pallas import tpu as pltpu
from jax.experimental.pallas import tpu_sc as plsc

N = 10000
D = 128
G = 64
NC = 2
NS = 16
NW = NC * NS
E_CHUNK = 128
N_PAD = 10240
ROWS_PER_TILE = N_PAD // NS
ROW_BLK = 1000
N_BLKS = N // ROW_BLK


def _combine_body(p_ref, h_ref, wrel_ref, wroot_ref, b_ref, out_ref):
    agg = p_ref[0] + p_ref[1]
    out = (jnp.dot(agg, wrel_ref[...], preferred_element_type=jnp.float32)
           + jnp.dot(h_ref[...], wroot_ref[...], preferred_element_type=jnp.float32)
           + b_ref[...])
    out_ref[...] = jnp.maximum(out, 0.0)


def _combine_matmul(p, h, wrel, wroot, b_row):
    return pl.pallas_call(
        _combine_body,
        grid=(N_BLKS,),
        in_specs=[
            pl.BlockSpec((NC, ROW_BLK, D), lambda i: (0, i, 0)),
            pl.BlockSpec((ROW_BLK, D), lambda i: (i, 0)),
            pl.BlockSpec((D, D), lambda i: (0, 0)),
            pl.BlockSpec((D, D), lambda i: (0, 0)),
            pl.BlockSpec((1, D), lambda i: (0, 0)),
        ],
        out_specs=pl.BlockSpec((ROW_BLK, D), lambda i: (i, 0)),
        out_shape=jax.ShapeDtypeStruct((N, D), jnp.float32),
    )(p, h, wrel, wroot, b_row)


def _final_body(p_ref, h_ref, wrel_ref, wroot_ref, b3_ref, batch_ref,
                wlin_ref, blin_ref, out_ref, sums, counts):
    i = pl.program_id(0)

    @pl.when(i == 0)
    def _():
        sums[...] = jnp.zeros_like(sums)
        counts[...] = jnp.zeros_like(counts)

    agg = p_ref[0] + p_ref[1]
    h = (jnp.dot(agg, wrel_ref[...], preferred_element_type=jnp.float32)
         + jnp.dot(h_ref[...], wroot_ref[...], preferred_element_type=jnp.float32)
         + b3_ref[...])
    b = batch_ref[0]
    gid = lax.broadcasted_iota(jnp.int32, (G, ROW_BLK), 0)
    onehot = (gid == b).astype(jnp.float32)
    sums[...] += jnp.dot(onehot, h, preferred_element_type=jnp.float32,
                         precision=lax.Precision.HIGHEST)
    counts[...] = counts[...] + jnp.sum(onehot, axis=1, keepdims=True)

    @pl.when(i == pl.num_programs(0) - 1)
    def _():
        pooled = sums[...] / jnp.maximum(counts[...], 1.0)
        out_ref[...] = jnp.dot(pooled, wlin_ref[...],
                               preferred_element_type=jnp.float32) + blin_ref[...]


def _final_pool(p, h, wrel, wroot, b3_row, batch3, wlin_pad, blin_row):
    return pl.pallas_call(
        _final_body,
        grid=(N_BLKS,),
        in_specs=[
            pl.BlockSpec((NC, ROW_BLK, D), lambda i: (0, i, 0)),
            pl.BlockSpec((ROW_BLK, D), lambda i: (i, 0)),
            pl.BlockSpec((D, D), lambda i: (0, 0)),
            pl.BlockSpec((D, D), lambda i: (0, 0)),
            pl.BlockSpec((1, D), lambda i: (0, 0)),
            pl.BlockSpec((1, 1, ROW_BLK), lambda i: (i, 0, 0)),
            pl.BlockSpec((D, D), lambda i: (0, 0)),
            pl.BlockSpec((1, D), lambda i: (0, 0)),
        ],
        out_specs=pl.BlockSpec((G, D), lambda i: (0, 0)),
        out_shape=jax.ShapeDtypeStruct((G, D), jnp.float32),
        scratch_shapes=[
            pltpu.VMEM((G, D), jnp.float32),
            pltpu.VMEM((G, D), jnp.float32),
        ],
        compiler_params=pltpu.CompilerParams(
            dimension_semantics=("arbitrary",)),
    )(p, h, wrel, wroot, b3_row, batch3, wlin_pad, blin_row)


def _sc_scatter(y, src, dst, zeros):
    E = src.shape[0]
    epw = E // NW
    n_chunks = epw // E_CHUNK
    tail = epw % E_CHUNK

    assert n_chunks >= 4 and (n_chunks - 2) % 4 == 0

    mesh = plsc.VectorSubcoreMesh(
        core_axis_name="c", subcore_axis_name="s",
        num_cores=NC, num_subcores=NS)

    @functools.partial(
        pl.kernel,
        out_type=jax.ShapeDtypeStruct((NC * N_PAD, D), jnp.float32),
        mesh=mesh,
        scratch_types=(
            [pltpu.VMEM((E_CHUNK,), jnp.int32)] * 8
            + [pltpu.VMEM((E_CHUNK, D), jnp.float32)] * 2
            + [pltpu.VMEM((max(tail, 8),), jnp.int32)] * 2
            + [pltpu.VMEM_SHARED((N_PAD, D), jnp.float32)]
            + [pltpu.SemaphoreType.DMA] * 6
        ),
    )
    def k(y_hbm, src_hbm, dst_hbm, zeros_hbm, out_hbm,
          s0, s1, s2, s3, d0, d1, d2, d3, r0, r1,
          src_t, dst_t, acc, i0, i1, i2, i3, g0, g1):
        srcs = [s0, s1, s2, s3]
        dsts = [d0, d1, d2, d3]
        rows = [r0, r1]
        si = [i0, i1, i2, i3]
        sg = [g0, g1]
        c = lax.axis_index("c")
        s = lax.axis_index("s")
        wid = c * NS + s
        row0 = s * ROWS_PER_TILE

        pltpu.sync_copy(zeros_hbm, acc.at[pl.ds(row0, ROWS_PER_TILE)])
        plsc.subcore_barrier()

        base0 = wid * epw

        def idx_start(i, u):
            b = base0 + i * E_CHUNK
            pltpu.async_copy(src_hbm.at[pl.ds(b, E_CHUNK)], srcs[u], si[u])
            pltpu.async_copy(dst_hbm.at[pl.ds(b, E_CHUNK)], dsts[u], si[u])

        def idx_wait(u):
            pltpu.make_async_copy(src_hbm.at[pl.ds(0, E_CHUNK)], srcs[u], si[u]).wait()
            pltpu.make_async_copy(dst_hbm.at[pl.ds(0, E_CHUNK)], dsts[u], si[u]).wait()

        def g_start(u, b):
            pltpu.async_copy(y_hbm.at[srcs[u]], rows[b], sg[b])

        def g_wait(u, b):
            pltpu.make_async_copy(y_hbm.at[srcs[u]], rows[b], sg[b]).wait()

        def scat(u, b):
            pltpu.sync_copy(rows[b], acc.at[dsts[u]], add=True)

        def stage(i, u, b, first, guard_ok):
            idx_wait(u)
            g_start(u, b)
            if not first:
                g_wait((u - 1) % 4, 1 - b)
                scat((u - 1) % 4, 1 - b)
            if guard_ok is None:
                idx_start(i + 2, (u + 2) % 4)
            elif guard_ok is not False:
                @pl.when(guard_ok)
                def _():
                    idx_start(i + 2, (u + 2) % 4)

        idx_start(0, 0)
        idx_start(1, 1)
        stage(0, 0, 0, True, None)
        stage(1, 1, 1, False, None)

        def body(q, carry):
            i0_ = 2 + 4 * q
            for u in range(4):
                i = i0_ + u
                ub = (2 + u) % 4
                stage(i, ub, u % 2, False, i + 2 < n_chunks)
            return carry

        lax.fori_loop(0, (n_chunks - 2) // 4, body, 0)

        g_wait((n_chunks - 1) % 4, (n_chunks - 1) % 2)
        scat((n_chunks - 1) % 4, (n_chunks - 1) % 2)

        if tail:
            tb = base0 + n_chunks * E_CHUNK
            pltpu.sync_copy(src_hbm.at[pl.ds(tb, tail)], src_t)
            pltpu.sync_copy(dst_hbm.at[pl.ds(tb, tail)], dst_t)
            rows_t = rows[0].at[pl.ds(0, tail)]
            pltpu.async_copy(y_hbm.at[src_t], rows_t, sg[0]).wait()
            pltpu.sync_copy(rows_t, acc.at[dst_t], add=True)

        plsc.subcore_barrier()

        pltpu.sync_copy(acc.at[pl.ds(row0, ROWS_PER_TILE)],
                        out_hbm.at[pl.ds(c * N_PAD + row0, ROWS_PER_TILE)])

    return k(y, src, dst, zeros)


def kernel(x, edge_index, batch, W1_rel, b1, W1_root, W2_rel, b2, W2_root,
           W3_rel, b3, W3_root, Wlin, blin):
    src = edge_index[0]
    dst = edge_index[1]
    zeros = jnp.zeros((ROWS_PER_TILE, D), jnp.float32)
    b1r = b1.reshape(1, D)
    b2r = b2.reshape(1, D)
    b3r = b3.reshape(1, D)
    batch3 = batch.reshape(N_BLKS, 1, ROW_BLK)
    wlin_pad = jnp.zeros((D, D), jnp.float32).at[:, : Wlin.shape[1]].set(Wlin)
    blin_row = jnp.zeros((1, D), jnp.float32).at[0, : blin.shape[0]].set(blin)

    p1 = _sc_scatter(x, src, dst, zeros).reshape(NC, N_PAD, D)
    h1 = _combine_matmul(p1, x, W1_rel, W1_root, b1r)
    p2 = _sc_scatter(h1, src, dst, zeros).reshape(NC, N_PAD, D)
    h2 = _combine_matmul(p2, h1, W2_rel, W2_root, b2r)
    p3 = _sc_scatter(h2, src, dst, zeros).reshape(NC, N_PAD, D)
    out = _final_pool(p3, h2, W3_rel, W3_root, b3r, batch3, wlin_pad, blin_row)
    return out[:, : Wlin.shape[1]]
